# Initial kernel scaffold; baseline (speedup 1.0000x reference)
import jax
import jax.numpy as jnp
from jax.experimental import pallas as pl


def kernel(x, edge_index, W_in, b_in, ln_g, ln_b, W1, b1, W2, b2, g_out, b_out, W_out, bo):
    raise NotImplementedError("write your pallas kernel here")


# speedup vs baseline: 8.8755x; 8.8755x over previous
import functools

import jax
import jax.numpy as jnp
from jax import lax
from jax.experimental import pallas as pl
from jax.experimental.pallas import tpu as pltpu
from jax.experimental.pallas import tpu_sc as plsc

NC = 2
NS = 16
NW = NC * NS
CHUNK = 128

_MESH = dict(core_axis_name="c", subcore_axis_name="s")


def _round_up(a, b):
    return (a + b - 1) // b * b


@functools.lru_cache(maxsize=None)
def _make_sc_deg(C, ND):
    CPW = C // NW
    RPS = ND // NS

    @functools.partial(
        pl.kernel,
        mesh=plsc.VectorSubcoreMesh(**_MESH),
        out_type=jax.ShapeDtypeStruct((NC, ND, 16), jnp.float32),
        scratch_types=[
            pltpu.VMEM((CHUNK,), jnp.int32),
            pltpu.VMEM((CHUNK, 16), jnp.float32),
            pltpu.VMEM((CHUNK, 16), jnp.float32),
            pltpu.VMEM_SHARED((ND, 16), jnp.float32),
        ],
    )
    def _deg(src_hbm, degp_hbm, idx_v, ones_v, zero_v, acc):
        c = lax.axis_index("c")
        s = lax.axis_index("s")
        w = s * NC + c

        def init_row(i, carry):
            ones_v[i] = jnp.full((16,), 1.0, jnp.float32)
            zero_v[i] = jnp.zeros((16,), jnp.float32)
            return carry

        lax.fori_loop(0, CHUNK, init_row, 0)
        for k in range(RPS // CHUNK):
            pltpu.sync_copy(zero_v, acc.at[pl.ds(s * RPS + k * CHUNK, CHUNK)])
        plsc.subcore_barrier()

        def body(i, carry):
            pltpu.sync_copy(src_hbm.at[w * CPW + i], idx_v)
            pltpu.sync_copy(ones_v, acc.at[idx_v], add=True)
            return carry

        lax.fori_loop(0, CPW, body, 0)
        plsc.subcore_barrier()
        pltpu.sync_copy(acc.at[pl.ds(s * RPS, RPS)],
                        degp_hbm.at[c, pl.ds(s * RPS, RPS)])

    return _deg


@functools.lru_cache(maxsize=None)
def _make_sc_agg(C, ND, Nr, Dd):
    CPW = C // NW
    RPS = ND // NS

    @functools.partial(
        pl.kernel,
        mesh=plsc.VectorSubcoreMesh(**_MESH),
        out_type=jax.ShapeDtypeStruct((NC, ND, Dd), jnp.float32),
        scratch_types=[
            pltpu.VMEM((CHUNK,), jnp.int32),
            pltpu.VMEM((CHUNK,), jnp.int32),
            pltpu.VMEM((CHUNK, Dd), jnp.float32),
            pltpu.VMEM((CHUNK, Dd), jnp.float32),
            pltpu.VMEM_SHARED((ND, Dd), jnp.float32),
            pltpu.SemaphoreType.DMA,
        ],
    )
    def _agg(rp_hbm, src_hbm, dst_hbm, out_hbm, sidx_v, didx_v, rows_v,
             zrow_v, acc, sem):
        c = lax.axis_index("c")
        s = lax.axis_index("s")
        w = s * NC + c

        def init_row(i, carry):
            for j in range(Dd // 16):
                zrow_v[i, pl.ds(j * 16, 16)] = jnp.zeros((16,), jnp.float32)
            return carry

        lax.fori_loop(0, CHUNK, init_row, 0)
        for k in range(RPS // CHUNK):
            pltpu.sync_copy(zrow_v, acc.at[pl.ds(s * RPS + k * CHUNK, CHUNK)])
        plsc.subcore_barrier()

        def body(i, carry):
            pltpu.sync_copy(src_hbm.at[w * CPW + i], sidx_v)
            pltpu.sync_copy(dst_hbm.at[w * CPW + i], didx_v)
            pltpu.async_copy(rp_hbm.at[sidx_v], rows_v, sem).wait()
            pltpu.sync_copy(rows_v, acc.at[didx_v], add=True)
            return carry

        lax.fori_loop(0, CPW, body, 0)
        plsc.subcore_barrier()
        pltpu.sync_copy(acc.at[pl.ds(s * RPS, RPS)],
                        out_hbm.at[c, pl.ds(s * RPS, RPS)])

    return _agg


def _ln(h, g, b):
    mu = jnp.mean(h, axis=-1, keepdims=True)
    var = jnp.mean((h - mu) ** 2, axis=-1, keepdims=True)
    return (h - mu) * lax.rsqrt(var + 1e-5) * g + b


def _dinv_from(degp_ref):
    deg = degp_ref[0, :, 0:1] + degp_ref[1, :, 0:1]
    return lax.rsqrt(jnp.maximum(deg, 1.0))


def _tc_in_body(x_ref, w_ref, b_ref, degp_ref, g_ref, lb_ref, h_ref, r_ref):
    h = jax.nn.gelu(jnp.dot(x_ref[...], w_ref[...],
                            preferred_element_type=jnp.float32) + b_ref[...])
    h_ref[...] = h
    r_ref[...] = _ln(h, g_ref[...], lb_ref[...]) * _dinv_from(degp_ref)


def _tc_mid_body(h_ref, yp_ref, degp_ref, w1_ref, b1_ref, w2_ref, b2_ref,
                 g_ref, lb_ref, hn_ref, rn_ref):
    dinv = _dinv_from(degp_ref)
    y = (yp_ref[0] + yp_ref[1]) * dinv
    t = jax.nn.gelu(jnp.dot(y, w1_ref[...],
                            preferred_element_type=jnp.float32) + b1_ref[...])
    t = jnp.dot(t, w2_ref[...], preferred_element_type=jnp.float32) + b2_ref[...]
    hn = h_ref[...] + t
    hn_ref[...] = hn
    rn_ref[...] = _ln(hn, g_ref[...], lb_ref[...]) * dinv


def _tc_out_body(h_ref, yp_ref, degp_ref, w1_ref, b1_ref, w2_ref, b2_ref,
                 g_ref, lb_ref, wo_ref, bo_ref, o_ref):
    dinv = _dinv_from(degp_ref)
    y = (yp_ref[0] + yp_ref[1]) * dinv
    t = jax.nn.gelu(jnp.dot(y, w1_ref[...],
                            preferred_element_type=jnp.float32) + b1_ref[...])
    t = jnp.dot(t, w2_ref[...], preferred_element_type=jnp.float32) + b2_ref[...]
    hn = h_ref[...] + t
    z = jnp.dot(_ln(hn, g_ref[...], lb_ref[...]), wo_ref[...],
                preferred_element_type=jnp.float32) + bo_ref[...]
    o_ref[...] = jax.nn.sigmoid(z)


def _row_spec(B, Dd):
    return pl.BlockSpec((B, Dd), lambda i: (i, 0))


def _full_spec(shape):
    nd = len(shape)
    return pl.BlockSpec(shape, lambda i, _n=nd: (0,) * _n)


def _degp_spec(B):
    return pl.BlockSpec((NC, B, 16), lambda i: (0, i, 0))


def _yp_spec(B, Dd):
    return pl.BlockSpec((NC, B, Dd), lambda i: (0, i, 0))


def _tc_in(x, W, b, degp, g, lb, B):
    Nn, Dd = x.shape
    grid = (Nn // B,)
    return pl.pallas_call(
        _tc_in_body,
        grid=grid,
        in_specs=[_row_spec(B, Dd), _full_spec(W.shape), _full_spec(b.shape),
                  _degp_spec(B), _full_spec(g.shape), _full_spec(lb.shape)],
        out_specs=[_row_spec(B, Dd), _row_spec(B, Dd)],
        out_shape=[jax.ShapeDtypeStruct((Nn, Dd), jnp.float32),
                   jax.ShapeDtypeStruct((Nn, Dd), jnp.float32)],
    )(x, W, b, degp, g, lb)


def _tc_mid(h, yp, degp, W1l, b1l, W2l, b2l, g, lb, B):
    Nn, Dd = h.shape
    grid = (Nn // B,)
    return pl.pallas_call(
        _tc_mid_body,
        grid=grid,
        in_specs=[_row_spec(B, Dd), _yp_spec(B, Dd), _degp_spec(B),
                  _full_spec(W1l.shape), _full_spec(b1l.shape),
                  _full_spec(W2l.shape), _full_spec(b2l.shape),
                  _full_spec(g.shape), _full_spec(lb.shape)],
        out_specs=[_row_spec(B, Dd), _row_spec(B, Dd)],
        out_shape=[jax.ShapeDtypeStruct((Nn, Dd), jnp.float32),
                   jax.ShapeDtypeStruct((Nn, Dd), jnp.float32)],
    )(h, yp, degp, W1l, b1l, W2l, b2l, g, lb)


def _tc_out(h, yp, degp, W1l, b1l, W2l, b2l, g, lb, Wo, bo, B):
    Nn, Dd = h.shape
    grid = (Nn // B,)
    return pl.pallas_call(
        _tc_out_body,
        grid=grid,
        in_specs=[_row_spec(B, Dd), _yp_spec(B, Dd), _degp_spec(B),
                  _full_spec(W1l.shape), _full_spec(b1l.shape),
                  _full_spec(W2l.shape), _full_spec(b2l.shape),
                  _full_spec(g.shape), _full_spec(lb.shape),
                  _full_spec(Wo.shape), _full_spec(bo.shape)],
        out_specs=[_row_spec(B, Dd)],
        out_shape=[jax.ShapeDtypeStruct((Nn, Dd), jnp.float32)],
    )(h, yp, degp, W1l, b1l, W2l, b2l, g, lb, Wo, bo)[0]


def kernel(x, edge_index, W_in, b_in, ln_g, ln_b, W1, b1, W2, b2,
           g_out, b_out, W_out, bo):
    Nn, Dd = x.shape
    E = edge_index.shape[1]
    L = W1.shape[0]
    ND = _round_up(Nn, NS * CHUNK)
    DUMMY = Nn
    B = 1000 if Nn % 1000 == 0 else 8

    src = edge_index[0]
    dst = edge_index[1]
    E_pad = _round_up(E, NW * CHUNK)
    pad = E_pad - E
    src_g = jnp.concatenate([src, jnp.zeros((pad,), jnp.int32)])
    dst_p = jnp.concatenate([dst, jnp.full((pad,), DUMMY, jnp.int32)])
    src_d = jnp.concatenate([src, jnp.full((pad,), DUMMY, jnp.int32)])
    src2d = src_g.reshape(-1, CHUNK)
    dst2d = dst_p.reshape(-1, CHUNK)
    srcd2d = src_d.reshape(-1, CHUNK)
    C = src2d.shape[0]

    degp = _make_sc_deg(C, ND)(srcd2d)

    h, r = _tc_in(x, W_in, b_in, degp, ln_g[0], ln_b[0], B)
    agg = _make_sc_agg(C, ND, Nn, Dd)
    for l in range(L):
        yp = agg(r, src2d, dst2d)
        if l + 1 < L:
            h, r = _tc_mid(h, yp, degp, W1[l], b1[l], W2[l], b2[l],
                           ln_g[l + 1], ln_b[l + 1], B)
        else:
            out = _tc_out(h, yp, degp, W1[l], b1[l], W2[l], b2[l],
                          g_out, b_out, W_out, bo, B)
    return out
